# baseline (speedup 1.0000x reference)
import functools

import jax
import jax.numpy as jnp
from jax import lax
from jax.experimental import pallas as pl
from jax.experimental.pallas import tpu as pltpu
from jax.experimental.pallas import tpu_sc as plsc

N = 10000
D = 256
L = 16
NPW = 16384
WR = 256
SENT = NPW - 1
TRASH = 256
TBLR = 264
TRASHP = TRASH
GRAN = 16
B = 64
EN = 163840
EC = 40960
CAPM = 16384

_mesh = lambda: plsc.VectorSubcoreMesh(core_axis_name="c", subcore_axis_name="s")


def _make_deg(E):
    ED = E // 32

    @functools.partial(
        pl.kernel,
        out_type=jax.ShapeDtypeStruct((32, NPW), jnp.float32),
        mesh=_mesh(),
        scratch_types=[
            pltpu.VMEM((ED,), jnp.int32),
            pltpu.VMEM((NPW + L,), jnp.float32),
        ],
    )
    def deg_k(dst_hbm, out_hbm, idx_v, tbl_v):
        c = lax.axis_index("c")
        s = lax.axis_index("s")
        t = c * 16 + s
        z = jnp.zeros((L,), jnp.float32)
        e0 = jnp.maximum(1 - lax.iota(jnp.int32, L), 0).astype(jnp.float32)

        def zz(i, _):
            tbl_v[pl.ds(i * L, L)] = z
            return 0

        lax.fori_loop(0, (NPW + L) // L, zz, 0)
        pltpu.sync_copy(dst_hbm.at[t], idx_v)

        def body(i, _):
            dv = idx_v[pl.ds(i * L, L)]
            for lane in range(L):
                d = dv[lane]
                tbl_v[pl.ds(d, L)] = tbl_v[pl.ds(d, L)] + e0
            return 0

        lax.fori_loop(0, ED // L, body, 0)
        pltpu.sync_copy(tbl_v.at[pl.ds(0, NPW)], out_hbm.at[t])

    return deg_k


_deg_n = _make_deg(EN)
_deg_c = _make_deg(EC)


def _make_bucketize(E):
    ED = E // 32
    CAPB = ED + GRAN
    CAPS = ED + 64 * (GRAN - 1) + GRAN
    CAPS = ((CAPS + L - 1) // L) * L

    @functools.partial(
        pl.kernel,
        out_type=(
            jax.ShapeDtypeStruct((32, 64, CAPB), jnp.int32),
            jax.ShapeDtypeStruct((32, 1024), jnp.int32),
        ),
        mesh=_mesh(),
        scratch_types=[
            pltpu.VMEM((ED,), jnp.int32),
            pltpu.VMEM((ED,), jnp.int32),
            pltpu.VMEM((1024,), jnp.int32),
            pltpu.VMEM((1024,), jnp.int32),
            pltpu.VMEM((1024,), jnp.int32),
            pltpu.VMEM((CAPS,), jnp.int32),
            pltpu.SemaphoreType.DMA,
        ],
    )
    def bk_k(src_hbm, dst_hbm, bkt_hbm, cnt_hbm, srcs_v, dsts_v, hist_v,
             starts_v, curs_v, stag_v, sem):
        c = lax.axis_index("c")
        s = lax.axis_index("s")
        t = c * 16 + s
        zi = jnp.zeros((L,), jnp.int32)
        tr = jnp.full((L,), TRASHP, jnp.int32)
        one = jnp.ones((L,), jnp.int32)

        pltpu.sync_copy(src_hbm.at[t], srcs_v)
        pltpu.sync_copy(dst_hbm.at[t], dsts_v)
        for o in range(64):
            hist_v[pl.ds(o * L, L)] = zi

        def count(i, _):
            ov = lax.shift_right_logical(dsts_v[pl.ds(i * L, L)], 8)
            for lane in range(L):
                o = ov[lane]
                hv = hist_v[pl.ds(o * L, L)]
                hist_v[pl.ds(o * L, L)] = hv + one
            return 0

        lax.fori_loop(0, ED // L, count, 0)

        acc = jnp.int32(0)
        for o in range(64):
            st = jnp.full((L,), acc, jnp.int32)
            starts_v[pl.ds(o * L, L)] = st
            curs_v[pl.ds(o * L, L)] = st
            cnt = hist_v[pl.ds(o * L, L)][0]
            acc = acc + ((cnt + (GRAN - 1)) // GRAN) * GRAN

        def mset(i, _):
            stag_v[pl.ds(i * L, L)] = tr
            return 0

        lax.fori_loop(0, CAPS // L, mset, 0)

        m0 = jnp.maximum(1 - lax.iota(jnp.int32, L), 0)
        m0n = 1 - m0

        def place(i, _):
            dv = dsts_v[pl.ds(i * L, L)]
            sv = srcs_v[pl.ds(i * L, L)]
            pk = lax.shift_left(sv, 9) | (dv & 255)
            ov = lax.shift_right_logical(dv, 8)
            for lane in range(L):
                o = ov[lane]
                cur = curs_v[pl.ds(o * L, L)][0]
                old = stag_v[pl.ds(cur, L)]
                stag_v[pl.ds(cur, L)] = old * m0n + pk[lane] * m0
                curs_v[pl.ds(o * L, L)] = jnp.full((L,), cur + 1, jnp.int32)
            return 0

        lax.fori_loop(0, ED // L, place, 0)

        def flsh(o, gtot):
            st = pl.multiple_of(starts_v[pl.ds(o * L, L)][0], GRAN)
            cnt = hist_v[pl.ds(o * L, L)][0]
            n16 = (cnt + (GRAN - 1)) // GRAN

            def fire(q, _):
                pltpu.async_copy(stag_v.at[pl.ds(st + q * GRAN, GRAN)],
                                 bkt_hbm.at[t, o, pl.ds(q * GRAN, GRAN)], sem)
                return 0

            lax.fori_loop(0, n16, fire, 0)
            return gtot + n16

        gtot = lax.fori_loop(0, 64, flsh, jnp.int32(0))

        def drain(q, _):
            pltpu.make_async_copy(stag_v.at[pl.ds(0, GRAN)],
                                  bkt_hbm.at[t, 0, pl.ds(0, GRAN)], sem).wait()
            return 0

        lax.fori_loop(0, gtot, drain, 0)
        pltpu.sync_copy(hist_v, cnt_hbm.at[t])

    return bk_k


_bkt_n = _make_bucketize(EN)
_bkt_c = _make_bucketize(EC)


def _make_agg(E):
    ED = E // 32
    CAPB = ED + GRAN

    @functools.partial(
        pl.kernel,
        out_type=jax.ShapeDtypeStruct((NPW, D), jnp.float32),
        mesh=_mesh(),
        scratch_types=[
            pltpu.VMEM((512,), jnp.int32),
            pltpu.VMEM((CAPM + 4 * L,), jnp.int32),
            pltpu.VMEM((B,), jnp.int32),
            pltpu.VMEM((B + L,), jnp.int32),
            pltpu.VMEM((B,), jnp.int32),
            pltpu.VMEM((B + L,), jnp.int32),
            pltpu.VMEM((B, D), jnp.float32),
            pltpu.VMEM((B, D), jnp.float32),
            pltpu.VMEM((TBLR, D), jnp.float32),
            pltpu.SemaphoreType.DMA,
            pltpu.SemaphoreType.DMA,
            pltpu.SemaphoreType.DMA,
        ],
    )
    def ag_k(y_hbm, bkt_hbm, cnt_hbm, out_hbm, cnts_v, mrg_v, idxa_v, lda_v,
             idxb_v, ldb_v, rowsa_v, rowsb_v, tbl_v, sem_m, sem_a, sem_b):
        c = lax.axis_index("c")
        s = lax.axis_index("s")
        t = c * 16 + s
        z = jnp.zeros((L,), jnp.float32)
        tr = jnp.full((L,), TRASHP, jnp.int32)

        def prep(b64, idx_v, ld_v):
            for q in range(B // L):
                pk = mrg_v[pl.ds(b64 * B + q * L, L)]
                idx_v[pl.ds(q * L, L)] = lax.shift_right_logical(pk, 9)
                ld_v[pl.ds(q * L, L)] = pk & 511

        def fire(idx_v, rows_v, sem):
            pltpu.async_copy(y_hbm.at[idx_v], rows_v, sem)

        def waitg(idx_v, rows_v, sem):
            pltpu.make_async_copy(y_hbm.at[idx_v], rows_v, sem).wait()

        def acc(ld_v, rows_v):
            def row(i, _):
                ld = ld_v[pl.ds(i, L)][0]
                for j in range(D // L):
                    cur = tbl_v[ld, pl.ds(j * L, L)]
                    tbl_v[ld, pl.ds(j * L, L)] = cur + rows_v[i, pl.ds(j * L, L)]
                return 0

            lax.fori_loop(0, B, row, 0)

        for widx in range(2):
            w = t * 2 + widx

            def ztbl(i, _):
                for j in range(D // L):
                    tbl_v[i, pl.ds(j * L, L)] = z
                return 0

            lax.fori_loop(0, TBLR, ztbl, 0)

            woff = pl.multiple_of(w * L, L)
            for sc in range(32):
                pltpu.async_copy(cnt_hbm.at[sc, pl.ds(woff, L)],
                                 cnts_v.at[pl.ds(sc * L, L)], sem_m)
            for sc in range(32):
                pltpu.make_async_copy(cnt_hbm.at[0, pl.ds(0, L)],
                                      cnts_v.at[pl.ds(0, L)], sem_m).wait()

            def group(g, _):
                mfill = jnp.int32(0)
                gcnt = jnp.int32(0)
                for j in range(3):
                    sc = g * 3 + j
                    live = sc < 32
                    sc = jnp.minimum(sc, 31)
                    cnt = cnts_v[pl.ds(sc * L, L)][0]
                    n16 = jnp.where(live, (cnt + (GRAN - 1)) // GRAN, 0)

                    def mfire(q, _, sc=sc, mf0=mfill):
                        pltpu.async_copy(
                            bkt_hbm.at[sc, w, pl.ds(q * GRAN, GRAN)],
                            mrg_v.at[pl.ds(pl.multiple_of(mf0, GRAN)
                                           + q * GRAN, GRAN)], sem_m)
                        return 0

                    lax.fori_loop(0, n16, mfire, 0)
                    mfill = mfill + n16 * GRAN
                    gcnt = gcnt + n16

                def mdrain(q, _):
                    pltpu.make_async_copy(
                        bkt_hbm.at[0, 0, pl.ds(0, GRAN)],
                        mrg_v.at[pl.ds(0, GRAN)], sem_m).wait()
                    return 0

                lax.fori_loop(0, gcnt, mdrain, 0)

                for j in range(4):
                    mrg_v[pl.ds(mfill + j * L, L)] = tr
                nb = (mfill + B - 1) // B

                @pl.when(nb > 0)
                def _():
                    prep(0, idxa_v, lda_v)
                    fire(idxa_v, rowsa_v, sem_a)

                @pl.when(nb > 1)
                def _():
                    prep(1, idxb_v, ldb_v)
                    fire(idxb_v, rowsb_v, sem_b)

                def pair(i2, _):
                    ba = i2 * 2

                    waitg(idxa_v, rowsa_v, sem_a)
                    acc(lda_v, rowsa_v)

                    @pl.when(ba + 2 < nb)
                    def _():
                        prep(ba + 2, idxa_v, lda_v)
                        fire(idxa_v, rowsa_v, sem_a)

                    @pl.when(ba + 1 < nb)
                    def _():
                        waitg(idxb_v, rowsb_v, sem_b)
                        acc(ldb_v, rowsb_v)

                    @pl.when(ba + 3 < nb)
                    def _():
                        prep(ba + 3, idxb_v, ldb_v)
                        fire(idxb_v, rowsb_v, sem_b)

                    return 0

                lax.fori_loop(0, (nb + 1) // 2, pair, 0)
                return 0

            lax.fori_loop(0, 11, group, 0)

            pltpu.sync_copy(tbl_v.at[pl.ds(0, WR)],
                            out_hbm.at[pl.ds(w * WR, WR)])

    return ag_k


_agg_n = _make_agg(EN)
_agg_c = _make_agg(EC)


BM = 512
_G = (N + BM - 1) // BM


def _dinv_body(dp_ref, out_ref):
    out_ref[...] = lax.rsqrt(jnp.sum(dp_ref[...], axis=0) + 1.0)


def _dinv(deg):
    return pl.pallas_call(
        _dinv_body,
        out_shape=jax.ShapeDtypeStruct((128, 128), jnp.float32),
    )(deg.reshape(32, 128, 128)).reshape(NPW, 1)


def _mm_body(h_ref, w_ref, dinv_ref, y_ref):
    y_ref[...] = jnp.dot(h_ref[...], w_ref[...],
                         preferred_element_type=jnp.float32) * dinv_ref[...]


def _mm_scale(h, W, dinv):
    return pl.pallas_call(
        _mm_body,
        grid=(_G,),
        in_specs=[
            pl.BlockSpec((BM, D), lambda i: (i, 0)),
            pl.BlockSpec((D, D), lambda i: (0, 0)),
            pl.BlockSpec((BM, 1), lambda i: (i, 0)),
        ],
        out_specs=pl.BlockSpec((BM, D), lambda i: (i, 0)),
        out_shape=jax.ShapeDtypeStruct((N, D), jnp.float32),
    )(h, W, dinv)


def _cmm_body(a_ref, y_ref, dp_ref, b_ref, w_ref, dc_ref, out_ref):
    h = jnp.maximum(
        dp_ref[...] * (a_ref[...] + y_ref[...]) + b_ref[...], 0.0)
    out_ref[...] = jnp.dot(h, w_ref[...],
                           preferred_element_type=jnp.float32) * dc_ref[...]


def _combine_mm(agg, y, dinv_p, b, W, dinv_c):
    return pl.pallas_call(
        _cmm_body,
        grid=(_G,),
        in_specs=[
            pl.BlockSpec((BM, D), lambda i: (i, 0)),
            pl.BlockSpec((BM, D), lambda i: (i, 0)),
            pl.BlockSpec((BM, 1), lambda i: (i, 0)),
            pl.BlockSpec((1, D), lambda i: (0, 0)),
            pl.BlockSpec((D, D), lambda i: (0, 0)),
            pl.BlockSpec((BM, 1), lambda i: (i, 0)),
        ],
        out_specs=pl.BlockSpec((BM, D), lambda i: (i, 0)),
        out_shape=jax.ShapeDtypeStruct((N, D), jnp.float32),
    )(agg, y, dinv_p, b.reshape(1, D), W, dinv_c)


def _fin_body(a_ref, y_ref, dp_ref, b_ref, out_ref):
    out_ref[...] = jnp.maximum(
        dp_ref[...] * (a_ref[...] + y_ref[...]) + b_ref[...], 0.0)


def _final(agg, y, dinv, b):
    return pl.pallas_call(
        _fin_body,
        grid=(_G,),
        in_specs=[
            pl.BlockSpec((BM, D), lambda i: (i, 0)),
            pl.BlockSpec((BM, D), lambda i: (i, 0)),
            pl.BlockSpec((BM, 1), lambda i: (i, 0)),
            pl.BlockSpec((1, D), lambda i: (0, 0)),
        ],
        out_specs=pl.BlockSpec((BM, D), lambda i: (i, 0)),
        out_shape=jax.ShapeDtypeStruct((N, D), jnp.float32),
    )(agg, y, dinv, b.reshape(1, D))


def _pad_edges(ei, E):
    pad = E - ei.shape[1]
    src = jnp.concatenate([ei[0], jnp.zeros((pad,), jnp.int32)])
    dst = jnp.concatenate([ei[1], jnp.full((pad,), SENT, jnp.int32)])
    return src.reshape(32, E // 32), dst.reshape(32, E // 32)


def kernel(x, edge_index, community_edge_index,
           W_n0, b_n0, W_c0, b_c0, W_n1, b_n1, W_c1, b_c1):
    src_n, dst_n = _pad_edges(edge_index, EN)
    src_c, dst_c = _pad_edges(community_edge_index, EC)

    dinv_n = _dinv(_deg_n(dst_n))
    dinv_c = _dinv(_deg_c(dst_c))
    bkt_n, cnt_n = _bkt_n(src_n, dst_n)
    bkt_c, cnt_c = _bkt_c(src_c, dst_c)

    y = _mm_scale(x, W_n0, dinv_n)
    agg = _agg_n(y, bkt_n, cnt_n)
    y = _combine_mm(agg, y, dinv_n, b_n0, W_c0, dinv_c)
    agg = _agg_c(y, bkt_c, cnt_c)
    y = _combine_mm(agg, y, dinv_c, b_c0, W_n1, dinv_n)
    agg = _agg_n(y, bkt_n, cnt_n)
    y = _combine_mm(agg, y, dinv_n, b_n1, W_c1, dinv_c)
    agg = _agg_c(y, bkt_c, cnt_c)
    return _final(agg, y, dinv_c, b_c1)


# speedup vs baseline: 1.4423x; 1.4423x over previous
import functools

import jax
import jax.numpy as jnp
from jax import lax
from jax.experimental import pallas as pl
from jax.experimental.pallas import tpu as pltpu
from jax.experimental.pallas import tpu_sc as plsc

N = 10000
D = 256
L = 16
NPW = 16384
WR = 256
SENT = NPW - 1
TRASH = 256
TBLR = 264
TRASHP = TRASH
GRAN = 16
B = 64
EN = 163840
EC = 40960
CAPM = 16384

_mesh = lambda: plsc.VectorSubcoreMesh(core_axis_name="c", subcore_axis_name="s")


def _make_deg(E):
    ED = E // 32

    @functools.partial(
        pl.kernel,
        out_type=jax.ShapeDtypeStruct((32, NPW), jnp.float32),
        mesh=_mesh(),
        scratch_types=[
            pltpu.VMEM((ED,), jnp.int32),
            pltpu.VMEM((NPW + L,), jnp.float32),
        ],
    )
    def deg_k(dst_hbm, out_hbm, idx_v, tbl_v):
        c = lax.axis_index("c")
        s = lax.axis_index("s")
        t = c * 16 + s
        z = jnp.zeros((L,), jnp.float32)
        e0 = jnp.maximum(1 - lax.iota(jnp.int32, L), 0).astype(jnp.float32)

        def zz(i, _):
            tbl_v[pl.ds(i * L, L)] = z
            return 0

        lax.fori_loop(0, (NPW + L) // L, zz, 0)
        pltpu.sync_copy(dst_hbm.at[t], idx_v)

        def body(i, _):
            dv = idx_v[pl.ds(i * L, L)]
            for lane in range(L):
                d = dv[lane]
                tbl_v[pl.ds(d, L)] = tbl_v[pl.ds(d, L)] + e0
            return 0

        lax.fori_loop(0, ED // L, body, 0)
        pltpu.sync_copy(tbl_v.at[pl.ds(0, NPW)], out_hbm.at[t])

    return deg_k


_deg_n = _make_deg(EN)
_deg_c = _make_deg(EC)


def _make_bucketize(E):
    ED = E // 32
    CAPB = ED + GRAN
    CAPS = ED + 64 * (GRAN - 1) + GRAN
    CAPS = ((CAPS + L - 1) // L) * L

    @functools.partial(
        pl.kernel,
        out_type=(
            jax.ShapeDtypeStruct((32, 64, CAPB), jnp.int32),
            jax.ShapeDtypeStruct((32, 1024), jnp.int32),
        ),
        mesh=_mesh(),
        scratch_types=[
            pltpu.VMEM((ED,), jnp.int32),
            pltpu.VMEM((ED,), jnp.int32),
            pltpu.VMEM((1024,), jnp.int32),
            pltpu.VMEM((1024,), jnp.int32),
            pltpu.VMEM((1024,), jnp.int32),
            pltpu.VMEM((CAPS,), jnp.int32),
            pltpu.SemaphoreType.DMA,
        ],
    )
    def bk_k(src_hbm, dst_hbm, bkt_hbm, cnt_hbm, srcs_v, dsts_v, hist_v,
             starts_v, curs_v, stag_v, sem):
        c = lax.axis_index("c")
        s = lax.axis_index("s")
        t = c * 16 + s
        zi = jnp.zeros((L,), jnp.int32)
        tr = jnp.full((L,), TRASHP, jnp.int32)
        one = jnp.ones((L,), jnp.int32)

        pltpu.sync_copy(src_hbm.at[t], srcs_v)
        pltpu.sync_copy(dst_hbm.at[t], dsts_v)
        for o in range(64):
            hist_v[pl.ds(o * L, L)] = zi

        def count(i, _):
            ov = lax.shift_right_logical(dsts_v[pl.ds(i * L, L)], 8)
            for lane in range(L):
                o = ov[lane]
                hv = hist_v[pl.ds(o * L, L)]
                hist_v[pl.ds(o * L, L)] = hv + one
            return 0

        lax.fori_loop(0, ED // L, count, 0)

        acc = jnp.int32(0)
        for o in range(64):
            st = jnp.full((L,), acc, jnp.int32)
            starts_v[pl.ds(o * L, L)] = st
            curs_v[pl.ds(o * L, L)] = st
            cnt = hist_v[pl.ds(o * L, L)][0]
            acc = acc + ((cnt + (GRAN - 1)) // GRAN) * GRAN

        def mset(i, _):
            stag_v[pl.ds(i * L, L)] = tr
            return 0

        lax.fori_loop(0, CAPS // L, mset, 0)

        m0 = jnp.maximum(1 - lax.iota(jnp.int32, L), 0)
        m0n = 1 - m0

        def place(i, _):
            dv = dsts_v[pl.ds(i * L, L)]
            sv = srcs_v[pl.ds(i * L, L)]
            pk = lax.shift_left(sv, 9) | (dv & 255)
            ov = lax.shift_right_logical(dv, 8)
            for lane in range(L):
                o = ov[lane]
                cur = curs_v[pl.ds(o * L, L)][0]
                old = stag_v[pl.ds(cur, L)]
                stag_v[pl.ds(cur, L)] = old * m0n + pk[lane] * m0
                curs_v[pl.ds(o * L, L)] = jnp.full((L,), cur + 1, jnp.int32)
            return 0

        lax.fori_loop(0, ED // L, place, 0)

        def flsh(o, gtot):
            st = pl.multiple_of(starts_v[pl.ds(o * L, L)][0], GRAN)
            cnt = hist_v[pl.ds(o * L, L)][0]
            n16 = (cnt + (GRAN - 1)) // GRAN

            def fire(q, _):
                pltpu.async_copy(stag_v.at[pl.ds(st + q * GRAN, GRAN)],
                                 bkt_hbm.at[t, o, pl.ds(q * GRAN, GRAN)], sem)
                return 0

            lax.fori_loop(0, n16, fire, 0)
            return gtot + n16

        gtot = lax.fori_loop(0, 64, flsh, jnp.int32(0))

        def drain(q, _):
            pltpu.make_async_copy(stag_v.at[pl.ds(0, GRAN)],
                                  bkt_hbm.at[t, 0, pl.ds(0, GRAN)], sem).wait()
            return 0

        lax.fori_loop(0, gtot, drain, 0)
        pltpu.sync_copy(hist_v, cnt_hbm.at[t])

    return bk_k


_bkt_n = _make_bucketize(EN)
_bkt_c = _make_bucketize(EC)


def _make_agg(E):
    ED = E // 32
    CAPB = ED + GRAN

    @functools.partial(
        pl.kernel,
        out_type=jax.ShapeDtypeStruct((NPW, D), jnp.float32),
        mesh=_mesh(),
        scratch_types=[
            pltpu.VMEM((512,), jnp.int32),
            pltpu.VMEM((CAPM + 4 * L,), jnp.int32),
            pltpu.VMEM((B,), jnp.int32),
            pltpu.VMEM((B + L,), jnp.int32),
            pltpu.VMEM((B,), jnp.int32),
            pltpu.VMEM((B + L,), jnp.int32),
            pltpu.VMEM((B, D), jnp.float32),
            pltpu.VMEM((B, D), jnp.float32),
            pltpu.VMEM((TBLR, D), jnp.float32),
            pltpu.SemaphoreType.DMA,
            pltpu.SemaphoreType.DMA,
            pltpu.SemaphoreType.DMA,
        ],
    )
    def ag_k(y_hbm, bkt_hbm, cnt_hbm, out_hbm, cnts_v, mrg_v, idxa_v, lda_v,
             idxb_v, ldb_v, rowsa_v, rowsb_v, tbl_v, sem_m, sem_a, sem_b):
        c = lax.axis_index("c")
        s = lax.axis_index("s")
        t = c * 16 + s
        z = jnp.zeros((L,), jnp.float32)
        tr = jnp.full((L,), TRASHP, jnp.int32)

        def prep(b64, idx_v, ld_v):
            for q in range(B // L):
                pk = mrg_v[pl.ds(b64 * B + q * L, L)]
                idx_v[pl.ds(q * L, L)] = lax.shift_right_logical(pk, 9)
                ld_v[pl.ds(q * L, L)] = pk & 511

        def fire(idx_v, rows_v, sem):
            pltpu.async_copy(y_hbm.at[idx_v], rows_v, sem)

        def waitg(idx_v, rows_v, sem):
            pltpu.make_async_copy(y_hbm.at[idx_v], rows_v, sem).wait()

        def acc(ld_v, rows_v):
            def row(i, _):
                ld = ld_v[pl.ds(i, L)][0]
                sums = [tbl_v[ld, pl.ds(j * L, L)] + rows_v[i, pl.ds(j * L, L)]
                        for j in range(D // L)]
                for j in range(D // L):
                    tbl_v[ld, pl.ds(j * L, L)] = sums[j]
                return 0

            lax.fori_loop(0, B, row, 0)

        for widx in range(2):
            w = t * 2 + widx

            def ztbl(i, _):
                for j in range(D // L):
                    tbl_v[i, pl.ds(j * L, L)] = z
                return 0

            lax.fori_loop(0, TBLR, ztbl, 0)

            woff = pl.multiple_of(w * L, L)
            for sc in range(32):
                pltpu.async_copy(cnt_hbm.at[sc, pl.ds(woff, L)],
                                 cnts_v.at[pl.ds(sc * L, L)], sem_m)
            for sc in range(32):
                pltpu.make_async_copy(cnt_hbm.at[0, pl.ds(0, L)],
                                      cnts_v.at[pl.ds(0, L)], sem_m).wait()

            def process(mfill):
                for j in range(4):
                    mrg_v[pl.ds(mfill + j * L, L)] = tr
                nb = (mfill + B - 1) // B

                @pl.when(nb > 0)
                def _():
                    prep(0, idxa_v, lda_v)
                    fire(idxa_v, rowsa_v, sem_a)

                @pl.when(nb > 1)
                def _():
                    prep(1, idxb_v, ldb_v)
                    fire(idxb_v, rowsb_v, sem_b)

                def pair(i2, _):
                    ba = i2 * 2

                    waitg(idxa_v, rowsa_v, sem_a)
                    acc(lda_v, rowsa_v)

                    @pl.when(ba + 2 < nb)
                    def _():
                        prep(ba + 2, idxa_v, lda_v)
                        fire(idxa_v, rowsa_v, sem_a)

                    @pl.when(ba + 1 < nb)
                    def _():
                        waitg(idxb_v, rowsb_v, sem_b)
                        acc(ldb_v, rowsb_v)

                        @pl.when(ba + 3 < nb)
                        def _():
                            prep(ba + 3, idxb_v, ldb_v)
                            fire(idxb_v, rowsb_v, sem_b)

                    return 0

                lax.fori_loop(0, (nb + 1) // 2, pair, 0)

            def mdrain_n(gc):
                def mdrain(q, _):
                    pltpu.make_async_copy(
                        bkt_hbm.at[0, 0, pl.ds(0, GRAN)],
                        mrg_v.at[pl.ds(0, GRAN)], sem_m).wait()
                    return 0

                lax.fori_loop(0, gc, mdrain, 0)

            def scbody(sc, carry):
                mfill, gcnt = carry
                cnt = cnts_v[pl.ds(sc * L, L)][0]
                n16 = (cnt + (GRAN - 1)) // GRAN
                over = mfill + n16 * GRAN > CAPM

                @pl.when(over)
                def _():
                    mdrain_n(gcnt)
                    process(mfill)

                mfill = jnp.where(over, 0, mfill)
                gcnt = jnp.where(over, 0, gcnt)

                def mfire(q, _):
                    pltpu.async_copy(
                        bkt_hbm.at[sc, w, pl.ds(q * GRAN, GRAN)],
                        mrg_v.at[pl.ds(pl.multiple_of(mfill, GRAN)
                                       + q * GRAN, GRAN)], sem_m)
                    return 0

                lax.fori_loop(0, n16, mfire, 0)
                return (mfill + n16 * GRAN, gcnt + n16)

            mfill, gcnt = lax.fori_loop(0, 32, scbody,
                                        (jnp.int32(0), jnp.int32(0)))
            mdrain_n(gcnt)
            process(mfill)

            pltpu.sync_copy(tbl_v.at[pl.ds(0, WR)],
                            out_hbm.at[pl.ds(w * WR, WR)])

    return ag_k


_agg_n = _make_agg(EN)
_agg_c = _make_agg(EC)


BM = 512
_G = (N + BM - 1) // BM


def _dinv_body(dp_ref, out_ref):
    out_ref[...] = lax.rsqrt(jnp.sum(dp_ref[...], axis=0) + 1.0)


def _dinv(deg):
    return pl.pallas_call(
        _dinv_body,
        out_shape=jax.ShapeDtypeStruct((128, 128), jnp.float32),
    )(deg.reshape(32, 128, 128)).reshape(NPW, 1)


def _mm_body(h_ref, w_ref, dinv_ref, y_ref):
    y_ref[...] = jnp.dot(h_ref[...], w_ref[...],
                         preferred_element_type=jnp.float32) * dinv_ref[...]


def _mm_scale(h, W, dinv):
    return pl.pallas_call(
        _mm_body,
        grid=(_G,),
        in_specs=[
            pl.BlockSpec((BM, D), lambda i: (i, 0)),
            pl.BlockSpec((D, D), lambda i: (0, 0)),
            pl.BlockSpec((BM, 1), lambda i: (i, 0)),
        ],
        out_specs=pl.BlockSpec((BM, D), lambda i: (i, 0)),
        out_shape=jax.ShapeDtypeStruct((N, D), jnp.float32),
    )(h, W, dinv)


def _cmm_body(a_ref, y_ref, dp_ref, b_ref, w_ref, dc_ref, out_ref):
    h = jnp.maximum(
        dp_ref[...] * (a_ref[...] + y_ref[...]) + b_ref[...], 0.0)
    out_ref[...] = jnp.dot(h, w_ref[...],
                           preferred_element_type=jnp.float32) * dc_ref[...]


def _combine_mm(agg, y, dinv_p, b, W, dinv_c):
    return pl.pallas_call(
        _cmm_body,
        grid=(_G,),
        in_specs=[
            pl.BlockSpec((BM, D), lambda i: (i, 0)),
            pl.BlockSpec((BM, D), lambda i: (i, 0)),
            pl.BlockSpec((BM, 1), lambda i: (i, 0)),
            pl.BlockSpec((1, D), lambda i: (0, 0)),
            pl.BlockSpec((D, D), lambda i: (0, 0)),
            pl.BlockSpec((BM, 1), lambda i: (i, 0)),
        ],
        out_specs=pl.BlockSpec((BM, D), lambda i: (i, 0)),
        out_shape=jax.ShapeDtypeStruct((N, D), jnp.float32),
    )(agg, y, dinv_p, b.reshape(1, D), W, dinv_c)


def _fin_body(a_ref, y_ref, dp_ref, b_ref, out_ref):
    out_ref[...] = jnp.maximum(
        dp_ref[...] * (a_ref[...] + y_ref[...]) + b_ref[...], 0.0)


def _final(agg, y, dinv, b):
    return pl.pallas_call(
        _fin_body,
        grid=(_G,),
        in_specs=[
            pl.BlockSpec((BM, D), lambda i: (i, 0)),
            pl.BlockSpec((BM, D), lambda i: (i, 0)),
            pl.BlockSpec((BM, 1), lambda i: (i, 0)),
            pl.BlockSpec((1, D), lambda i: (0, 0)),
        ],
        out_specs=pl.BlockSpec((BM, D), lambda i: (i, 0)),
        out_shape=jax.ShapeDtypeStruct((N, D), jnp.float32),
    )(agg, y, dinv, b.reshape(1, D))


def _pad_edges(ei, E):
    pad = E - ei.shape[1]
    src = jnp.concatenate([ei[0], jnp.zeros((pad,), jnp.int32)])
    dst = jnp.concatenate([ei[1], jnp.full((pad,), SENT, jnp.int32)])
    return src.reshape(32, E // 32), dst.reshape(32, E // 32)


def kernel(x, edge_index, community_edge_index,
           W_n0, b_n0, W_c0, b_c0, W_n1, b_n1, W_c1, b_c1):
    src_n, dst_n = _pad_edges(edge_index, EN)
    src_c, dst_c = _pad_edges(community_edge_index, EC)

    dinv_n = _dinv(_deg_n(dst_n))
    dinv_c = _dinv(_deg_c(dst_c))
    bkt_n, cnt_n = _bkt_n(src_n, dst_n)
    bkt_c, cnt_c = _bkt_c(src_c, dst_c)

    y = _mm_scale(x, W_n0, dinv_n)
    agg = _agg_n(y, bkt_n, cnt_n)
    y = _combine_mm(agg, y, dinv_n, b_n0, W_c0, dinv_c)
    agg = _agg_c(y, bkt_c, cnt_c)
    y = _combine_mm(agg, y, dinv_c, b_c0, W_n1, dinv_n)
    agg = _agg_n(y, bkt_n, cnt_n)
    y = _combine_mm(agg, y, dinv_n, b_n1, W_c1, dinv_c)
    agg = _agg_c(y, bkt_c, cnt_c)
    return _final(agg, y, dinv_c, b_c1)


# speedup vs baseline: 1.4801x; 1.0262x over previous
import functools

import jax
import jax.numpy as jnp
from jax import lax
from jax.experimental import pallas as pl
from jax.experimental.pallas import tpu as pltpu
from jax.experimental.pallas import tpu_sc as plsc

N = 10000
D = 256
L = 16
NPW = 16384
WR = 256
SENT = NPW - 1
TRASH = 256
TBLR = 264
TRASHP = TRASH
GRAN = 16
B = 64
EN = 163840
EC = 40960
CAPM = 16384

_mesh = lambda: plsc.VectorSubcoreMesh(core_axis_name="c", subcore_axis_name="s")


def _make_deg(E):
    ED = E // 32

    @functools.partial(
        pl.kernel,
        out_type=jax.ShapeDtypeStruct((32, NPW), jnp.float32),
        mesh=_mesh(),
        scratch_types=[
            pltpu.VMEM((ED,), jnp.int32),
            pltpu.VMEM((NPW + L,), jnp.float32),
        ],
    )
    def deg_k(dst_hbm, out_hbm, idx_v, tbl_v):
        c = lax.axis_index("c")
        s = lax.axis_index("s")
        t = c * 16 + s
        z = jnp.zeros((L,), jnp.float32)
        e0 = jnp.maximum(1 - lax.iota(jnp.int32, L), 0).astype(jnp.float32)

        def zz(i, _):
            tbl_v[pl.ds(i * L, L)] = z
            return 0

        lax.fori_loop(0, (NPW + L) // L, zz, 0)
        pltpu.sync_copy(dst_hbm.at[t], idx_v)

        def body(i, _):
            dv = idx_v[pl.ds(i * L, L)]
            for lane in range(L):
                d = dv[lane]
                tbl_v[pl.ds(d, L)] = tbl_v[pl.ds(d, L)] + e0
            return 0

        lax.fori_loop(0, ED // L, body, 0)
        pltpu.sync_copy(tbl_v.at[pl.ds(0, NPW)], out_hbm.at[t])

    return deg_k


_deg_n = _make_deg(EN)
_deg_c = _make_deg(EC)


def _make_bucketize(E):
    ED = E // 32
    CAPB = ED + GRAN
    CAPS = ED + 64 * (GRAN - 1) + GRAN
    CAPS = ((CAPS + L - 1) // L) * L

    @functools.partial(
        pl.kernel,
        out_type=(
            jax.ShapeDtypeStruct((32, 64, CAPB), jnp.int32),
            jax.ShapeDtypeStruct((32, 1024), jnp.int32),
        ),
        mesh=_mesh(),
        scratch_types=[
            pltpu.VMEM((ED,), jnp.int32),
            pltpu.VMEM((ED,), jnp.int32),
            pltpu.VMEM((1024,), jnp.int32),
            pltpu.VMEM((1024,), jnp.int32),
            pltpu.VMEM((1024,), jnp.int32),
            pltpu.VMEM((CAPS,), jnp.int32),
            pltpu.SemaphoreType.DMA,
        ],
    )
    def bk_k(src_hbm, dst_hbm, bkt_hbm, cnt_hbm, srcs_v, dsts_v, hist_v,
             starts_v, curs_v, stag_v, sem):
        c = lax.axis_index("c")
        s = lax.axis_index("s")
        t = c * 16 + s
        zi = jnp.zeros((L,), jnp.int32)
        tr = jnp.full((L,), TRASHP, jnp.int32)
        one = jnp.ones((L,), jnp.int32)

        pltpu.sync_copy(src_hbm.at[t], srcs_v)
        pltpu.sync_copy(dst_hbm.at[t], dsts_v)
        for o in range(64):
            hist_v[pl.ds(o * L, L)] = zi

        def count(i, _):
            ov = lax.shift_right_logical(dsts_v[pl.ds(i * L, L)], 8)
            for lane in range(L):
                o = ov[lane]
                hv = hist_v[pl.ds(o * L, L)]
                hist_v[pl.ds(o * L, L)] = hv + one
            return 0

        lax.fori_loop(0, ED // L, count, 0)

        acc = jnp.int32(0)
        for o in range(64):
            st = jnp.full((L,), acc, jnp.int32)
            starts_v[pl.ds(o * L, L)] = st
            curs_v[pl.ds(o * L, L)] = st
            cnt = hist_v[pl.ds(o * L, L)][0]
            acc = acc + ((cnt + (GRAN - 1)) // GRAN) * GRAN

        def mset(i, _):
            stag_v[pl.ds(i * L, L)] = tr
            return 0

        lax.fori_loop(0, CAPS // L, mset, 0)

        m0 = jnp.maximum(1 - lax.iota(jnp.int32, L), 0)
        m0n = 1 - m0

        def place(i, _):
            dv = dsts_v[pl.ds(i * L, L)]
            sv = srcs_v[pl.ds(i * L, L)]
            pk = lax.shift_left(sv, 9) | (dv & 255)
            ov = lax.shift_right_logical(dv, 8)
            for lane in range(L):
                o = ov[lane]
                cur = curs_v[pl.ds(o * L, L)][0]
                old = stag_v[pl.ds(cur, L)]
                stag_v[pl.ds(cur, L)] = old * m0n + pk[lane] * m0
                curs_v[pl.ds(o * L, L)] = jnp.full((L,), cur + 1, jnp.int32)
            return 0

        lax.fori_loop(0, ED // L, place, 0)

        def flsh(o, gtot):
            st = pl.multiple_of(starts_v[pl.ds(o * L, L)][0], GRAN)
            cnt = hist_v[pl.ds(o * L, L)][0]
            n16 = (cnt + (GRAN - 1)) // GRAN

            def fire(q, _):
                pltpu.async_copy(stag_v.at[pl.ds(st + q * GRAN, GRAN)],
                                 bkt_hbm.at[t, o, pl.ds(q * GRAN, GRAN)], sem)
                return 0

            lax.fori_loop(0, n16, fire, 0)
            return gtot + n16

        gtot = lax.fori_loop(0, 64, flsh, jnp.int32(0))

        def drain(q, _):
            pltpu.make_async_copy(stag_v.at[pl.ds(0, GRAN)],
                                  bkt_hbm.at[t, 0, pl.ds(0, GRAN)], sem).wait()
            return 0

        lax.fori_loop(0, gtot, drain, 0)
        pltpu.sync_copy(hist_v, cnt_hbm.at[t])

    return bk_k


_bkt_n = _make_bucketize(EN)
_bkt_c = _make_bucketize(EC)


def _make_agg(E):
    ED = E // 32
    CAPB = ED + GRAN

    @functools.partial(
        pl.kernel,
        out_type=jax.ShapeDtypeStruct((NPW, D), jnp.float32),
        mesh=_mesh(),
        scratch_types=[
            pltpu.VMEM((512,), jnp.int32),
            pltpu.VMEM((CAPM + 4 * L,), jnp.int32),
            pltpu.VMEM((B,), jnp.int32),
            pltpu.VMEM((B + L,), jnp.int32),
            pltpu.VMEM((B,), jnp.int32),
            pltpu.VMEM((B + L,), jnp.int32),
            pltpu.VMEM((B, D), jnp.float32),
            pltpu.VMEM((B, D), jnp.float32),
            pltpu.VMEM((TBLR, D), jnp.float32),
            pltpu.SemaphoreType.DMA,
            pltpu.SemaphoreType.DMA,
            pltpu.SemaphoreType.DMA,
        ],
    )
    def ag_k(y_hbm, bkt_hbm, cnt_hbm, out_hbm, cnts_v, mrg_v, idxa_v, lda_v,
             idxb_v, ldb_v, rowsa_v, rowsb_v, tbl_v, sem_m, sem_a, sem_b):
        c = lax.axis_index("c")
        s = lax.axis_index("s")
        t = c * 16 + s
        z = jnp.zeros((L,), jnp.float32)
        tr = jnp.full((L,), TRASHP, jnp.int32)

        def prep(b64, idx_v, ld_v):
            for q in range(B // L):
                pk = mrg_v[pl.ds(b64 * B + q * L, L)]
                idx_v[pl.ds(q * L, L)] = lax.shift_right_logical(pk, 9)
                ld_v[pl.ds(q * L, L)] = pk & 511

        def fire(idx_v, rows_v, sem):
            pltpu.async_copy(y_hbm.at[idx_v], rows_v, sem)

        def waitg(idx_v, rows_v, sem):
            pltpu.make_async_copy(y_hbm.at[idx_v], rows_v, sem).wait()

        def acc(ld_v, rows_v):
            def row2(i2, _):
                for u in range(2):
                    i = i2 * 2 + u
                    ld = ld_v[pl.ds(i, L)][0]
                    sums = [tbl_v[ld, pl.ds(j * L, L)]
                            + rows_v[i, pl.ds(j * L, L)]
                            for j in range(D // L)]
                    for j in range(D // L):
                        tbl_v[ld, pl.ds(j * L, L)] = sums[j]
                return 0

            lax.fori_loop(0, B // 2, row2, 0)

        for widx in range(2):
            w = t * 2 + widx

            def ztbl(i, _):
                for j in range(D // L):
                    tbl_v[i, pl.ds(j * L, L)] = z
                return 0

            lax.fori_loop(0, TBLR, ztbl, 0)

            woff = pl.multiple_of(w * L, L)
            for sc in range(32):
                pltpu.async_copy(cnt_hbm.at[sc, pl.ds(woff, L)],
                                 cnts_v.at[pl.ds(sc * L, L)], sem_m)
            for sc in range(32):
                pltpu.make_async_copy(cnt_hbm.at[0, pl.ds(0, L)],
                                      cnts_v.at[pl.ds(0, L)], sem_m).wait()

            def process(mfill):
                for j in range(4):
                    mrg_v[pl.ds(mfill + j * L, L)] = tr
                nb = (mfill + B - 1) // B

                @pl.when(nb > 0)
                def _():
                    prep(0, idxa_v, lda_v)
                    fire(idxa_v, rowsa_v, sem_a)

                @pl.when(nb > 1)
                def _():
                    prep(1, idxb_v, ldb_v)
                    fire(idxb_v, rowsb_v, sem_b)

                def pair(i2, _):
                    ba = i2 * 2

                    waitg(idxa_v, rowsa_v, sem_a)
                    acc(lda_v, rowsa_v)

                    @pl.when(ba + 2 < nb)
                    def _():
                        prep(ba + 2, idxa_v, lda_v)
                        fire(idxa_v, rowsa_v, sem_a)

                    @pl.when(ba + 1 < nb)
                    def _():
                        waitg(idxb_v, rowsb_v, sem_b)
                        acc(ldb_v, rowsb_v)

                        @pl.when(ba + 3 < nb)
                        def _():
                            prep(ba + 3, idxb_v, ldb_v)
                            fire(idxb_v, rowsb_v, sem_b)

                    return 0

                lax.fori_loop(0, (nb + 1) // 2, pair, 0)

            def mdrain_n(gc):
                def mdrain(q, _):
                    pltpu.make_async_copy(
                        bkt_hbm.at[0, 0, pl.ds(0, GRAN)],
                        mrg_v.at[pl.ds(0, GRAN)], sem_m).wait()
                    return 0

                lax.fori_loop(0, gc, mdrain, 0)

            def scbody(sc, carry):
                mfill, gcnt = carry
                cnt = cnts_v[pl.ds(sc * L, L)][0]
                n16 = (cnt + (GRAN - 1)) // GRAN
                over = mfill + n16 * GRAN > CAPM

                @pl.when(over)
                def _():
                    mdrain_n(gcnt)
                    process(mfill)

                mfill = jnp.where(over, 0, mfill)
                gcnt = jnp.where(over, 0, gcnt)

                def mfire(q, _):
                    pltpu.async_copy(
                        bkt_hbm.at[sc, w, pl.ds(q * GRAN, GRAN)],
                        mrg_v.at[pl.ds(pl.multiple_of(mfill, GRAN)
                                       + q * GRAN, GRAN)], sem_m)
                    return 0

                lax.fori_loop(0, n16, mfire, 0)
                return (mfill + n16 * GRAN, gcnt + n16)

            mfill, gcnt = lax.fori_loop(0, 32, scbody,
                                        (jnp.int32(0), jnp.int32(0)))
            mdrain_n(gcnt)
            process(mfill)

            pltpu.sync_copy(tbl_v.at[pl.ds(0, WR)],
                            out_hbm.at[pl.ds(w * WR, WR)])

    return ag_k


_agg_n = _make_agg(EN)
_agg_c = _make_agg(EC)


BM = 512
_G = (N + BM - 1) // BM


def _dinv_body(dp_ref, out_ref):
    out_ref[...] = lax.rsqrt(jnp.sum(dp_ref[...], axis=0) + 1.0)


def _dinv(deg):
    return pl.pallas_call(
        _dinv_body,
        out_shape=jax.ShapeDtypeStruct((128, 128), jnp.float32),
    )(deg.reshape(32, 128, 128)).reshape(NPW, 1)


def _mm_body(h_ref, w_ref, dinv_ref, y_ref):
    y_ref[...] = jnp.dot(h_ref[...], w_ref[...],
                         preferred_element_type=jnp.float32) * dinv_ref[...]


def _mm_scale(h, W, dinv):
    return pl.pallas_call(
        _mm_body,
        grid=(_G,),
        in_specs=[
            pl.BlockSpec((BM, D), lambda i: (i, 0)),
            pl.BlockSpec((D, D), lambda i: (0, 0)),
            pl.BlockSpec((BM, 1), lambda i: (i, 0)),
        ],
        out_specs=pl.BlockSpec((BM, D), lambda i: (i, 0)),
        out_shape=jax.ShapeDtypeStruct((N, D), jnp.float32),
    )(h, W, dinv)


def _cmm_body(a_ref, y_ref, dp_ref, b_ref, w_ref, dc_ref, out_ref):
    h = jnp.maximum(
        dp_ref[...] * (a_ref[...] + y_ref[...]) + b_ref[...], 0.0)
    out_ref[...] = jnp.dot(h, w_ref[...],
                           preferred_element_type=jnp.float32) * dc_ref[...]


def _combine_mm(agg, y, dinv_p, b, W, dinv_c):
    return pl.pallas_call(
        _cmm_body,
        grid=(_G,),
        in_specs=[
            pl.BlockSpec((BM, D), lambda i: (i, 0)),
            pl.BlockSpec((BM, D), lambda i: (i, 0)),
            pl.BlockSpec((BM, 1), lambda i: (i, 0)),
            pl.BlockSpec((1, D), lambda i: (0, 0)),
            pl.BlockSpec((D, D), lambda i: (0, 0)),
            pl.BlockSpec((BM, 1), lambda i: (i, 0)),
        ],
        out_specs=pl.BlockSpec((BM, D), lambda i: (i, 0)),
        out_shape=jax.ShapeDtypeStruct((N, D), jnp.float32),
    )(agg, y, dinv_p, b.reshape(1, D), W, dinv_c)


def _fin_body(a_ref, y_ref, dp_ref, b_ref, out_ref):
    out_ref[...] = jnp.maximum(
        dp_ref[...] * (a_ref[...] + y_ref[...]) + b_ref[...], 0.0)


def _final(agg, y, dinv, b):
    return pl.pallas_call(
        _fin_body,
        grid=(_G,),
        in_specs=[
            pl.BlockSpec((BM, D), lambda i: (i, 0)),
            pl.BlockSpec((BM, D), lambda i: (i, 0)),
            pl.BlockSpec((BM, 1), lambda i: (i, 0)),
            pl.BlockSpec((1, D), lambda i: (0, 0)),
        ],
        out_specs=pl.BlockSpec((BM, D), lambda i: (i, 0)),
        out_shape=jax.ShapeDtypeStruct((N, D), jnp.float32),
    )(agg, y, dinv, b.reshape(1, D))


def _pad_edges(ei, E):
    pad = E - ei.shape[1]
    src = jnp.concatenate([ei[0], jnp.zeros((pad,), jnp.int32)])
    dst = jnp.concatenate([ei[1], jnp.full((pad,), SENT, jnp.int32)])
    return src.reshape(32, E // 32), dst.reshape(32, E // 32)


def kernel(x, edge_index, community_edge_index,
           W_n0, b_n0, W_c0, b_c0, W_n1, b_n1, W_c1, b_c1):
    src_n, dst_n = _pad_edges(edge_index, EN)
    src_c, dst_c = _pad_edges(community_edge_index, EC)

    dinv_n = _dinv(_deg_n(dst_n))
    dinv_c = _dinv(_deg_c(dst_c))
    bkt_n, cnt_n = _bkt_n(src_n, dst_n)
    bkt_c, cnt_c = _bkt_c(src_c, dst_c)

    y = _mm_scale(x, W_n0, dinv_n)
    agg = _agg_n(y, bkt_n, cnt_n)
    y = _combine_mm(agg, y, dinv_n, b_n0, W_c0, dinv_c)
    agg = _agg_c(y, bkt_c, cnt_c)
    y = _combine_mm(agg, y, dinv_c, b_c0, W_n1, dinv_n)
    agg = _agg_n(y, bkt_n, cnt_n)
    y = _combine_mm(agg, y, dinv_n, b_n1, W_c1, dinv_c)
    agg = _agg_c(y, bkt_c, cnt_c)
    return _final(agg, y, dinv_c, b_c1)
